# baseline (speedup 1.0000x reference)
import functools

import jax
import jax.numpy as jnp
from jax import lax
from jax.experimental import pallas as pl
from jax.experimental.pallas import tpu as pltpu
from jax.experimental.pallas import tpu_sc as plsc


LANES = 16


@functools.lru_cache(maxsize=None)
def _make_pool(B, S, E, NC, NS):
    NW = NC * NS
    BPW = B // NW
    NJ = 2
    SH = S // NJ
    EC = E // LANES
    mesh = plsc.VectorSubcoreMesh(core_axis_name="c", subcore_axis_name="s")

    @functools.partial(
        pl.kernel,
        out_type=jax.ShapeDtypeStruct((B, E), jnp.float32),
        mesh=mesh,
        scratch_types=[
            pltpu.VMEM((BPW, NJ, SH), jnp.int32),
            pltpu.VMEM((S, E), jnp.float32),
            pltpu.VMEM((BPW, E), jnp.float32),
            pltpu.SemaphoreType.DMA,
        ],
        compiler_params=pltpu.CompilerParams(use_tc_tiling_on_sc=False),
    )
    def pool(table_hbm, idx_hbm, out_hbm, idx_v, buf_v, pool_v, sem):
        wid = lax.axis_index("s") * NC + lax.axis_index("c")
        base = wid * BPW
        pltpu.sync_copy(idx_hbm.at[pl.ds(base, BPW)], idx_v)

        @pl.loop(0, BPW)
        def _row(i):
            copies = [
                pltpu.async_copy(
                    table_hbm.at[idx_v.at[i, j]],
                    buf_v.at[pl.ds(j * SH, SH)],
                    sem,
                )
                for j in range(NJ)
            ]
            for c in copies:
                c.wait()

            def body(s, accs):
                return tuple(
                    accs[c] + buf_v[s, pl.ds(c * LANES, LANES)]
                    for c in range(EC)
                )

            accs = lax.fori_loop(
                0, S, body,
                tuple(jnp.zeros((LANES,), jnp.float32) for _ in range(EC)),
                unroll=8,
            )
            for c in range(EC):
                pool_v[i, pl.ds(c * LANES, LANES)] = accs[c] * (1.0 / S)

        pltpu.sync_copy(pool_v, out_hbm.at[pl.ds(base, BPW)])

    return pool


def _dense_body(p_ref, w1_ref, b1_ref, w2_ref, b2_ref, o_ref):
    h = jnp.dot(p_ref[...], w1_ref[...], preferred_element_type=jnp.float32)
    h = h + b1_ref[...]
    o = jnp.dot(h, w2_ref[...], preferred_element_type=jnp.float32)
    o = o + b2_ref[...]
    m = jnp.max(o, axis=1, keepdims=True)
    e = jnp.exp(o - m)
    o_ref[...] = e / jnp.sum(e, axis=1, keepdims=True)


def kernel(input, table, W1, b1, W2, b2):
    B, S = input.shape
    E = table.shape[1]
    H = W1.shape[0]
    C = W2.shape[0]
    info = plsc.get_sparse_core_info()
    pool = _make_pool(B, S, E, info.num_cores, info.num_subcores)
    idx = input.reshape(B, 2, S // 2)
    pooled = pool(table, idx)
    return pl.pallas_call(
        _dense_body,
        out_shape=jax.ShapeDtypeStruct((B, C), jnp.float32),
    )(pooled, W1.T, b1.reshape(1, H), W2.T, b2.reshape(1, C))
